# baseline (device time: 118591 ns/iter reference)
import jax
import jax.numpy as jnp
from jax import lax
from jax.experimental import pallas as pl
from jax.experimental.pallas import tpu as pltpu

N_DEV = 4


def kernel(x, w_mat, scale_x, scale_w):
    m_per, k = x.shape
    _, n_per = w_mat.shape
    half = m_per // 2
    m_full = N_DEV * m_per
    n_tile = 512

    def body(x_ref, w_ref, sx_ref, sw_ref, out_ref,
             x8_ref, comm_ref, w8_ref, wstage_ref, ostage_ref,
             send_sems, recv_sems, wsems, osems):
        my = lax.axis_index("i")
        left = lax.rem(my - 1 + N_DEV, N_DEV)
        right = lax.rem(my + 1, N_DEV)


        w_dma_tile = n_tile // 2

        def w_dma(j, slot):
            return pltpu.make_async_copy(
                w_ref.at[:, j * w_dma_tile:(j + 1) * w_dma_tile],
                wstage_ref.at[slot],
                wsems.at[slot],
            )

        w_dma(0, 0).start()

        x8_ref[...] = x_ref[...].astype(jnp.float8_e5m2)

        send_r = pltpu.make_async_remote_copy(
            src_ref=x8_ref, dst_ref=comm_ref.at[0],
            send_sem=send_sems.at[0], recv_sem=recv_sems.at[0],
            device_id=(right,), device_id_type=pl.DeviceIdType.MESH,
        )
        send_l = pltpu.make_async_remote_copy(
            src_ref=x8_ref, dst_ref=comm_ref.at[1],
            send_sem=send_sems.at[1], recv_sem=recv_sems.at[1],
            device_id=(left,), device_id_type=pl.DeviceIdType.MESH,
        )
        send_r.start()
        send_l.start()

        scale = sx_ref[0] * sw_ref[0]

        ocopy_inflight = [None, None]
        ocopy_count = [0, 0]

        def gemm_tile(origin, data, j):
            acc = lax.dot_general(
                data, w8_ref[:, j * n_tile:(j + 1) * n_tile],
                (((1,), (0,)), ((), ())),
                preferred_element_type=jnp.float32,
            )
            slot = ocopy_count[0] % 2
            ocopy_count[0] += 1
            if ocopy_inflight[slot] is not None:
                ocopy_inflight[slot].wait()
            ostage_ref[slot] = jnp.maximum(acc * scale, 0.0)
            cp = pltpu.make_async_copy(
                ostage_ref.at[slot],
                out_ref.at[pl.ds(origin * m_per, m_per),
                           j * n_tile:(j + 1) * n_tile],
                osems.at[slot],
            )
            cp.start()
            ocopy_inflight[slot] = cp

        def gemm(origin, data):
            for j in range(n_per // n_tile):
                gemm_tile(origin, data, j)

        n_w = n_per // w_dma_tile
        for j in range(n_per // n_tile):
            for h in range(2):
                jw = 2 * j + h
                if jw + 1 < n_w:
                    w_dma(jw + 1, (jw + 1) % 2).start()
                w_dma(jw, jw % 2).wait()
                w8_ref[:, jw * w_dma_tile:(jw + 1) * w_dma_tile] = (
                    wstage_ref[jw % 2].astype(jnp.float8_e5m2)
                )
            gemm_tile(my, x8_ref[...], j)

        send_r.wait()
        send_l.wait()

        fwd_r = pltpu.make_async_remote_copy(
            src_ref=comm_ref.at[0, :half], dst_ref=comm_ref.at[2, :half],
            send_sem=send_sems.at[2], recv_sem=recv_sems.at[2],
            device_id=(right,), device_id_type=pl.DeviceIdType.MESH,
        )
        fwd_l = pltpu.make_async_remote_copy(
            src_ref=comm_ref.at[1, half:], dst_ref=comm_ref.at[2, half:],
            send_sem=send_sems.at[3], recv_sem=recv_sems.at[3],
            device_id=(left,), device_id_type=pl.DeviceIdType.MESH,
        )
        fwd_r.start()
        fwd_l.start()

        gemm(left, comm_ref[0])
        gemm(right, comm_ref[1])

        fwd_r.wait()
        fwd_l.wait()

        opp = lax.rem(my + 2, N_DEV)
        gemm(opp, comm_ref[2])

        for cp in ocopy_inflight:
            if cp is not None:
                cp.wait()

    return pl.pallas_call(
        body,
        out_shape=jax.ShapeDtypeStruct((m_full, n_per), jnp.float32),
        in_specs=[
            pl.BlockSpec(memory_space=pltpu.VMEM),
            pl.BlockSpec(memory_space=pltpu.MemorySpace.HBM),
            pl.BlockSpec(memory_space=pltpu.SMEM),
            pl.BlockSpec(memory_space=pltpu.SMEM),
        ],
        out_specs=pl.BlockSpec(memory_space=pltpu.MemorySpace.HBM),
        scratch_shapes=[
            pltpu.VMEM((m_per, k), jnp.float8_e5m2),
            pltpu.VMEM((3, m_per, k), jnp.float8_e5m2),
            pltpu.VMEM((k, n_per), jnp.float8_e5m2),
            pltpu.VMEM((2, k, n_tile // 2), jnp.float32),
            pltpu.VMEM((2, m_per, n_tile), jnp.float32),
            pltpu.SemaphoreType.DMA((4,)),
            pltpu.SemaphoreType.DMA((4,)),
            pltpu.SemaphoreType.DMA((2,)),
            pltpu.SemaphoreType.DMA((2,)),
        ],
        compiler_params=pltpu.CompilerParams(
            vmem_limit_bytes=63 * 1024 * 1024,
        ),
    )(x, w_mat, scale_x, scale_w)


# device time: 109170 ns/iter; 1.0863x vs baseline; 1.0863x over previous
import jax
import jax.numpy as jnp
from jax import lax
from jax.experimental import pallas as pl
from jax.experimental.pallas import tpu as pltpu

N_DEV = 4


def kernel(x, w_mat, scale_x, scale_w):
    m_per, k = x.shape
    _, n_per = w_mat.shape
    half = m_per // 2
    quart = m_per // 4
    m_full = N_DEV * m_per
    n_tile = 512

    def body(x_ref, w_ref, sx_ref, sw_ref, out_ref,
             x8_ref, comm_ref, w8_ref, wstage_ref, ostage_ref,
             send_sems, recv_sems, wsems, osems):
        my = lax.axis_index("i")
        left = lax.rem(my - 1 + N_DEV, N_DEV)
        right = lax.rem(my + 1, N_DEV)
        opp = lax.rem(my + 2, N_DEV)

        barrier_sem = pltpu.get_barrier_semaphore()
        for nbr in (left, right):
            pl.semaphore_signal(
                barrier_sem, inc=1,
                device_id=(nbr,), device_id_type=pl.DeviceIdType.MESH,
            )
        pl.semaphore_wait(barrier_sem, 2)

        w_dma_tile = n_tile // 2

        def w_dma(j, slot):
            return pltpu.make_async_copy(
                w_ref.at[:, j * w_dma_tile:(j + 1) * w_dma_tile],
                wstage_ref.at[slot],
                wsems.at[slot],
            )

        w_dma(0, 0).start()

        x8_ref[...] = x_ref[...].astype(jnp.float8_e5m2)

        def rdma(i, src, dst, target):
            return pltpu.make_async_remote_copy(
                src_ref=src, dst_ref=dst,
                send_sem=send_sems.at[i], recv_sem=recv_sems.at[i],
                device_id=(target,), device_id_type=pl.DeviceIdType.MESH,
            )

        a_rt = rdma(0, x8_ref.at[:half], comm_ref.at[0, :half], right)
        a_rb = rdma(1, x8_ref.at[half:], comm_ref.at[0, half:], right)
        a_lb = rdma(2, x8_ref.at[half:], comm_ref.at[1, half:], left)
        a_lt = rdma(3, x8_ref.at[:half], comm_ref.at[1, :half], left)
        a_rt.start()
        a_lb.start()
        a_rb.start()
        a_lt.start()

        scale = sx_ref[0] * sw_ref[0]

        ocopy_inflight = [None, None]
        ocopy_count = [0]

        def gemm_tiles(row_start, data, j_list):
            for j in j_list:
                acc = lax.dot_general(
                    data, w8_ref[:, j * n_tile:(j + 1) * n_tile],
                    (((1,), (0,)), ((), ())),
                    preferred_element_type=jnp.float32,
                )
                rows = data.shape[0]
                slot = ocopy_count[0] % 2
                ocopy_count[0] += 1
                if ocopy_inflight[slot] is not None:
                    ocopy_inflight[slot].wait()
                ostage_ref[slot, :rows] = jnp.maximum(acc * scale, 0.0)
                cp = pltpu.make_async_copy(
                    ostage_ref.at[slot, :rows],
                    out_ref.at[pl.ds(row_start, rows),
                               j * n_tile:(j + 1) * n_tile],
                    osems.at[slot],
                )
                cp.start()
                ocopy_inflight[slot] = cp

        def gemm(row_start, data):
            gemm_tiles(row_start, data, range(n_per // n_tile))

        n_w = n_per // w_dma_tile
        for j in range(n_per // n_tile):
            for h in range(2):
                jw = 2 * j + h
                if jw + 1 < n_w:
                    w_dma(jw + 1, (jw + 1) % 2).start()
                w_dma(jw, jw % 2).wait()
                w8_ref[:, jw * w_dma_tile:(jw + 1) * w_dma_tile] = (
                    wstage_ref[jw % 2].astype(jnp.float8_e5m2)
                )
            gemm_tiles(my * m_per, x8_ref[...], [j])

        a_rt.wait_recv()
        a_lb.wait_recv()
        f_r1 = rdma(4, comm_ref.at[0, :quart], comm_ref.at[2, :quart], right)
        f_r2 = rdma(5, comm_ref.at[0, quart:half], comm_ref.at[2, quart:half],
                    right)
        f_l1 = rdma(6, comm_ref.at[1, half:half + quart],
                    comm_ref.at[2, half:half + quart], left)
        f_l2 = rdma(7, comm_ref.at[1, half + quart:],
                    comm_ref.at[2, half + quart:], left)
        f_r1.start()
        f_r2.start()
        f_l1.start()
        f_l2.start()

        gemm(left * m_per, comm_ref[0, :half])
        gemm(right * m_per + half, comm_ref[1, half:])

        a_rb.wait_recv()
        a_lt.wait_recv()
        gemm(left * m_per + half, comm_ref[0, half:])
        gemm(right * m_per, comm_ref[1, :half])

        f_r1.wait_recv()
        f_l1.wait_recv()
        gemm(opp * m_per, comm_ref[2, :quart])
        gemm(opp * m_per + half, comm_ref[2, half:half + quart])

        f_r2.wait_recv()
        f_l2.wait_recv()
        gemm(opp * m_per + quart, comm_ref[2, quart:half])
        gemm(opp * m_per + half + quart, comm_ref[2, half + quart:])

        for r in (a_rt, a_rb, a_lb, a_lt, f_r1, f_r2, f_l1, f_l2):
            r.wait_send()
        for cp in ocopy_inflight:
            if cp is not None:
                cp.wait()

    return pl.pallas_call(
        body,
        out_shape=jax.ShapeDtypeStruct((m_full, n_per), jnp.float32),
        in_specs=[
            pl.BlockSpec(memory_space=pltpu.VMEM),
            pl.BlockSpec(memory_space=pltpu.MemorySpace.HBM),
            pl.BlockSpec(memory_space=pltpu.SMEM),
            pl.BlockSpec(memory_space=pltpu.SMEM),
        ],
        out_specs=pl.BlockSpec(memory_space=pltpu.MemorySpace.HBM),
        scratch_shapes=[
            pltpu.VMEM((m_per, k), jnp.float8_e5m2),
            pltpu.VMEM((3, m_per, k), jnp.float8_e5m2),
            pltpu.VMEM((k, n_per), jnp.float8_e5m2),
            pltpu.VMEM((2, k, n_tile // 2), jnp.float32),
            pltpu.VMEM((2, m_per, n_tile), jnp.float32),
            pltpu.SemaphoreType.DMA((8,)),
            pltpu.SemaphoreType.DMA((8,)),
            pltpu.SemaphoreType.DMA((2,)),
            pltpu.SemaphoreType.DMA((2,)),
        ],
        compiler_params=pltpu.CompilerParams(
            collective_id=0,
            vmem_limit_bytes=63 * 1024 * 1024,
        ),
    )(x, w_mat, scale_x, scale_w)


# device time: 108368 ns/iter; 1.0943x vs baseline; 1.0074x over previous
import jax
import jax.numpy as jnp
from jax import lax
from jax.experimental import pallas as pl
from jax.experimental.pallas import tpu as pltpu

N_DEV = 4


def kernel(x, w_mat, scale_x, scale_w):
    m_per, k = x.shape
    _, n_per = w_mat.shape
    half = m_per // 2
    quart = m_per // 4
    m_full = N_DEV * m_per
    n_tile = 512

    def body(x_ref, w_ref, sx_ref, sw_ref, out_ref,
             x8_ref, comm_ref, w8_ref, wstage_ref, ostage_ref,
             send_sems, recv_sems, wsems, osems):
        my = lax.axis_index("i")
        left = lax.rem(my - 1 + N_DEV, N_DEV)
        right = lax.rem(my + 1, N_DEV)
        opp = lax.rem(my + 2, N_DEV)

        barrier_sem = pltpu.get_barrier_semaphore()
        for nbr in (left, right):
            pl.semaphore_signal(
                barrier_sem, inc=1,
                device_id=(nbr,), device_id_type=pl.DeviceIdType.MESH,
            )

        w_dma_tile = n_tile // 2

        def w_dma(j, slot):
            return pltpu.make_async_copy(
                w_ref.at[:, j * w_dma_tile:(j + 1) * w_dma_tile],
                wstage_ref.at[slot],
                wsems.at[slot],
            )

        w_dma(0, 0).start()

        x8_ref[...] = x_ref[...].astype(jnp.float8_e5m2)

        pl.semaphore_wait(barrier_sem, 2)

        def rdma(i, src, dst, target):
            return pltpu.make_async_remote_copy(
                src_ref=src, dst_ref=dst,
                send_sem=send_sems.at[i], recv_sem=recv_sems.at[i],
                device_id=(target,), device_id_type=pl.DeviceIdType.MESH,
            )

        a_rt = rdma(0, x8_ref.at[:half], comm_ref.at[0, :half], right)
        a_rb = rdma(1, x8_ref.at[half:], comm_ref.at[0, half:], right)
        a_lb = rdma(2, x8_ref.at[half:], comm_ref.at[1, half:], left)
        a_lt = rdma(3, x8_ref.at[:half], comm_ref.at[1, :half], left)
        a_rt.start()
        a_lb.start()
        a_rb.start()
        a_lt.start()

        scale = sx_ref[0] * sw_ref[0]

        ocopy_inflight = [None, None]
        ocopy_count = [0]

        def gemm_tiles(row_start, data, j_list):
            for j in j_list:
                acc = lax.dot_general(
                    data, w8_ref[:, j * n_tile:(j + 1) * n_tile],
                    (((1,), (0,)), ((), ())),
                    preferred_element_type=jnp.float32,
                )
                rows = data.shape[0]
                slot = ocopy_count[0] % 2
                ocopy_count[0] += 1
                if ocopy_inflight[slot] is not None:
                    ocopy_inflight[slot].wait()
                ostage_ref[slot, :rows] = jnp.maximum(acc * scale, 0.0)
                cp = pltpu.make_async_copy(
                    ostage_ref.at[slot, :rows],
                    out_ref.at[pl.ds(row_start, rows),
                               j * n_tile:(j + 1) * n_tile],
                    osems.at[slot],
                )
                cp.start()
                ocopy_inflight[slot] = cp

        def gemm(row_start, data):
            gemm_tiles(row_start, data, range(n_per // n_tile))

        n_w = n_per // w_dma_tile
        for j in range(n_per // n_tile):
            for h in range(2):
                jw = 2 * j + h
                if jw + 1 < n_w:
                    w_dma(jw + 1, (jw + 1) % 2).start()
                w_dma(jw, jw % 2).wait()
                w8_ref[:, jw * w_dma_tile:(jw + 1) * w_dma_tile] = (
                    wstage_ref[jw % 2].astype(jnp.float8_e5m2)
                )
            gemm_tiles(my * m_per, x8_ref[...], [j])

        a_rt.wait_recv()
        a_lb.wait_recv()
        f_r1 = rdma(4, comm_ref.at[0, :quart], comm_ref.at[2, :quart], right)
        f_r2 = rdma(5, comm_ref.at[0, quart:half], comm_ref.at[2, quart:half],
                    right)
        f_l1 = rdma(6, comm_ref.at[1, half:half + quart],
                    comm_ref.at[2, half:half + quart], left)
        f_l2 = rdma(7, comm_ref.at[1, half + quart:],
                    comm_ref.at[2, half + quart:], left)
        f_r1.start()
        f_r2.start()
        f_l1.start()
        f_l2.start()

        gemm(left * m_per, comm_ref[0, :half])
        gemm(right * m_per + half, comm_ref[1, half:])

        a_rb.wait_recv()
        a_lt.wait_recv()
        gemm(left * m_per + half, comm_ref[0, half:])
        gemm(right * m_per, comm_ref[1, :half])

        f_r1.wait_recv()
        f_l1.wait_recv()
        gemm(opp * m_per, comm_ref[2, :quart])
        gemm(opp * m_per + half, comm_ref[2, half:half + quart])

        f_r2.wait_recv()
        f_l2.wait_recv()
        gemm(opp * m_per + quart, comm_ref[2, quart:half])
        gemm(opp * m_per + half + quart, comm_ref[2, half + quart:])

        for r in (a_rt, a_rb, a_lb, a_lt, f_r1, f_r2, f_l1, f_l2):
            r.wait_send()
        for cp in ocopy_inflight:
            if cp is not None:
                cp.wait()

    return pl.pallas_call(
        body,
        out_shape=jax.ShapeDtypeStruct((m_full, n_per), jnp.float32),
        in_specs=[
            pl.BlockSpec(memory_space=pltpu.VMEM),
            pl.BlockSpec(memory_space=pltpu.MemorySpace.HBM),
            pl.BlockSpec(memory_space=pltpu.SMEM),
            pl.BlockSpec(memory_space=pltpu.SMEM),
        ],
        out_specs=pl.BlockSpec(memory_space=pltpu.MemorySpace.HBM),
        scratch_shapes=[
            pltpu.VMEM((m_per, k), jnp.float8_e5m2),
            pltpu.VMEM((3, m_per, k), jnp.float8_e5m2),
            pltpu.VMEM((k, n_per), jnp.float8_e5m2),
            pltpu.VMEM((2, k, n_tile // 2), jnp.float32),
            pltpu.VMEM((2, m_per, n_tile), jnp.float32),
            pltpu.SemaphoreType.DMA((8,)),
            pltpu.SemaphoreType.DMA((8,)),
            pltpu.SemaphoreType.DMA((2,)),
            pltpu.SemaphoreType.DMA((2,)),
        ],
        compiler_params=pltpu.CompilerParams(
            collective_id=0,
            vmem_limit_bytes=63 * 1024 * 1024,
        ),
    )(x, w_mat, scale_x, scale_w)
